# Initial kernel scaffold; baseline (speedup 1.0000x reference)
import jax
import jax.numpy as jnp
from jax.experimental import pallas as pl


def kernel(X, W1, b1, W2, b2, node_idx, edge_idx):
    raise NotImplementedError("write your pallas kernel here")


# speedup vs baseline: 7.0703x; 7.0703x over previous
import functools

import jax
import jax.numpy as jnp
from jax import lax
from jax.experimental import pallas as pl
from jax.experimental.pallas import tpu as pltpu
from jax.experimental.pallas import tpu_sc as plsc

N = 10000
M = 10000
NNZ = 160000
D = 256
DH = 128
NC = 2
NT = 16
K = 128
PT = 10240
CH = PT // K
NNZ_PAD = NT * PT
SACC = 10240
ZR = 32
ROWS = 1000
HPT = NNZ // NT


def _mesh():
    return plsc.VectorSubcoreMesh(core_axis_name="c", subcore_axis_name="s")


@functools.cache
def _build_sc_degrees():
    @functools.partial(
        pl.kernel,
        out_type=(
            jax.ShapeDtypeStruct((NT, N), jnp.float32),
            jax.ShapeDtypeStruct((NT, M), jnp.float32),
        ),
        mesh=_mesh(),
        compiler_params=pltpu.CompilerParams(needs_layout_passes=False),
        scratch_types=[
            pltpu.VMEM((HPT,), jnp.int32),
            pltpu.VMEM((N,), jnp.float32),
        ],
    )
    def sc_degrees(node_hbm, edge_hbm, dvp_hbm, dep_hbm, idx_v, hist_v):
        c = lax.axis_index("c")
        t = lax.axis_index("s")

        def do_hist(src_hbm, out_hbm):
            pltpu.sync_copy(src_hbm.at[pl.ds(t * HPT, HPT)], idx_v)

            def zero(i, carry):
                hist_v[pl.ds(i * 16, 16)] = jnp.zeros((16,), jnp.float32)
                return carry

            lax.fori_loop(0, N // 16, zero, 0)

            ones = jnp.ones((16,), jnp.float32)

            def acc(i, carry):
                idx = idx_v[pl.ds(i * 16, 16)]
                plsc.addupdate_scatter(hist_v, [idx], ones)
                return carry

            lax.fori_loop(0, HPT // 16, acc, 0)
            pltpu.sync_copy(hist_v, out_hbm.at[t])

        @pl.when(c == 0)
        def _():
            do_hist(node_hbm, dvp_hbm)

        @pl.when(c == 1)
        def _():
            do_hist(edge_hbm, dep_hbm)

    return sc_degrees


def _sc_degrees_call(node_idx, edge_idx):
    return _build_sc_degrees()(node_idx, edge_idx)


@functools.cache
def _build_sc_spmm():
    @functools.partial(
        pl.kernel,
        out_type=jax.ShapeDtypeStruct((NC, M, DH), jnp.float32),
        mesh=_mesh(),
        compiler_params=pltpu.CompilerParams(needs_layout_passes=False),
        scratch_types=[
            pltpu.VMEM((2, 2, K), jnp.int32),
            pltpu.VMEM((2, K, DH), jnp.float32),
            pltpu.VMEM((ZR, DH), jnp.float32),
            pltpu.VMEM_SHARED((SACC, DH), jnp.float32),
            pltpu.SemaphoreType.DMA,
            pltpu.SemaphoreType.DMA,
        ],
    )
    def sc_spmm(table_hbm, pairs_hbm, out_hbm, pbuf, rows, zbuf, acc, sem0, sem1):
        c = lax.axis_index("c")
        t = lax.axis_index("s")
        tbl = table_hbm.at[c]
        sems = (sem0, sem1)

        def zset(i, carry):
            r = i // (DH // 16)
            col = (i % (DH // 16)) * 16
            zbuf[r, pl.ds(col, 16)] = jnp.zeros((16,), jnp.float32)
            return carry

        lax.fori_loop(0, ZR * (DH // 16), zset, 0)

        rows_per_tile = SACC // NT

        def zacc(i, carry):
            pltpu.sync_copy(zbuf, acc.at[pl.ds(t * rows_per_tile + i * ZR, ZR)])
            return carry

        lax.fori_loop(0, rows_per_tile // ZR, zacc, 0)
        plsc.subcore_barrier()

        def load_pairs(j, slot):
            pltpu.sync_copy(pairs_hbm.at[t * CH + j], pbuf.at[slot])

        def start_gather(slot):
            pltpu.async_copy(tbl.at[pbuf.at[slot, 0]], rows.at[slot], sems[slot])

        def wait_gather(slot):
            pltpu.make_async_copy(
                tbl.at[pbuf.at[slot, 0]], rows.at[slot], sems[slot]
            ).wait()

        def scatter_add(slot):
            pltpu.sync_copy(rows.at[slot], acc.at[pbuf.at[slot, 1]], add=True)

        load_pairs(0, 0)
        start_gather(0)

        def body(jj, carry):
            load_pairs(2 * jj + 1, 1)
            start_gather(1)
            wait_gather(0)
            scatter_add(0)

            @pl.when(jj != CH // 2 - 1)
            def _():
                load_pairs(2 * jj + 2, 0)
                start_gather(0)

            wait_gather(1)
            scatter_add(1)
            return carry

        lax.fori_loop(0, CH // 2, body, 0)
        plsc.subcore_barrier()

        wr = 624

        @pl.when(t != NT - 1)
        def _():
            pltpu.sync_copy(
                acc.at[pl.ds(t * wr, wr)],
                out_hbm.at[c].at[pl.ds(t * wr, wr)],
            )

        @pl.when(t == NT - 1)
        def _():
            pltpu.sync_copy(
                acc.at[pl.ds((NT - 1) * wr, M - (NT - 1) * wr)],
                out_hbm.at[c].at[pl.ds((NT - 1) * wr, M - (NT - 1) * wr)],
            )

    return sc_spmm


def _sc_spmm_call(table, pairs):
    return _build_sc_spmm()(table, pairs)


def _degrees_body(dvp_ref, dep_ref, a_ref, einv_ref):
    dv = jnp.sum(dvp_ref[...], axis=0, keepdims=True)
    de = jnp.sum(dep_ref[...], axis=0, keepdims=True)
    a_ref[...] = lax.rsqrt(dv)
    einv_ref[...] = 1.0 / de


def _tc_degrees(dvp, dep):
    return pl.pallas_call(
        _degrees_body,
        out_shape=[
            jax.ShapeDtypeStruct((1, N), jnp.float32),
            jax.ShapeDtypeStruct((1, M), jnp.float32),
        ],
    )(dvp, dep)


def _mm1_body(x_ref, w_ref, b_ref, a_ref, out_ref):
    y = lax.dot_general(
        x_ref[...], w_ref[...], (((1,), (1,)), ((), ())),
        preferred_element_type=jnp.float32,
    )
    out_ref[...] = ((y + b_ref[...]) * a_ref[...])[None]


def _tc_mm1(X, W1, b1, acol):
    return pl.pallas_call(
        _mm1_body,
        grid=(NC, N // ROWS),
        in_specs=[
            pl.BlockSpec((ROWS, D), lambda c, i: (i, 0)),
            pl.BlockSpec((DH, D), lambda c, i: (c, 0)),
            pl.BlockSpec((1, DH), lambda c, i: (0, c)),
            pl.BlockSpec((ROWS, 1), lambda c, i: (i, 0)),
        ],
        out_specs=pl.BlockSpec((1, ROWS, DH), lambda c, i: (c, i, 0)),
        out_shape=jax.ShapeDtypeStruct((NC, N, DH), jnp.float32),
    )(X, W1, b1.reshape(1, D), acol)


def _mid_body(z_ref, e_ref, out_ref):
    out_ref[...] = (z_ref[0] * e_ref[...])[None]


def _tc_mid(Z, einv_col):
    return pl.pallas_call(
        _mid_body,
        grid=(NC, M // ROWS),
        in_specs=[
            pl.BlockSpec((1, ROWS, DH), lambda c, i: (c, i, 0)),
            pl.BlockSpec((ROWS, 1), lambda c, i: (i, 0)),
        ],
        out_specs=pl.BlockSpec((1, ROWS, DH), lambda c, i: (c, i, 0)),
        out_shape=jax.ShapeDtypeStruct((NC, M, DH), jnp.float32),
    )(Z, einv_col)


def _mm2_body(z_ref, a_ref, w_ref, b_ref, out_ref):
    k = pl.program_id(2)
    a = a_ref[...]
    h = jnp.maximum(z_ref[0] * a, 0.0)
    p = lax.dot_general(
        h, w_ref[...], (((1,), (1,)), ((), ())),
        preferred_element_type=jnp.float32,
    )

    @pl.when(k == 0)
    def _():
        out_ref[...] = p[None]

    @pl.when(k == 1)
    def _():
        out_ref[...] = ((out_ref[0] + p + b_ref[...]) * a)[None]


def _tc_mm2(Zv, W2, b2, acol):
    return pl.pallas_call(
        _mm2_body,
        grid=(NC, N // ROWS, NC),
        in_specs=[
            pl.BlockSpec((1, ROWS, DH), lambda c, i, k: (k, i, 0)),
            pl.BlockSpec((ROWS, 1), lambda c, i, k: (i, 0)),
            pl.BlockSpec((DH, DH), lambda c, i, k: (c, k)),
            pl.BlockSpec((1, DH), lambda c, i, k: (0, c)),
        ],
        out_specs=pl.BlockSpec((1, ROWS, DH), lambda c, i, k: (c, i, 0)),
        out_shape=jax.ShapeDtypeStruct((NC, N, DH), jnp.float32),
    )(Zv, acol, W2, b2.reshape(1, D))


def _final_body(z_ref, a_ref, out_ref):
    out_ref[...] = z_ref[0] * a_ref[...]


def _tc_final(Zv, acol):
    return pl.pallas_call(
        _final_body,
        grid=(NC, N // ROWS),
        in_specs=[
            pl.BlockSpec((1, ROWS, DH), lambda c, i: (c, i, 0)),
            pl.BlockSpec((ROWS, 1), lambda c, i: (i, 0)),
        ],
        out_specs=pl.BlockSpec((ROWS, DH), lambda c, i: (i, c)),
        out_shape=jax.ShapeDtypeStruct((N, D), jnp.float32),
    )(Zv, acol)


def _pack_pairs(gidx, sidx, gmod, spad_base):
    npad = NNZ_PAD - NNZ
    fill = jnp.arange(npad, dtype=jnp.int32)
    g = jnp.concatenate([gidx.astype(jnp.int32), fill % gmod])
    s = jnp.concatenate([sidx.astype(jnp.int32),
                         spad_base + fill % (SACC - spad_base)])
    g3 = g.reshape(NT, CH, 1, K)
    s3 = s.reshape(NT, CH, 1, K)
    return jnp.concatenate([g3, s3], axis=2).reshape(NT * CH, 2, K)


def kernel(X, W1, b1, W2, b2, node_idx, edge_idx):
    pairs_ne = _pack_pairs(node_idx, edge_idx, N, M)
    pairs_en = _pack_pairs(edge_idx, node_idx, M, N)

    dvp, dep = _sc_degrees_call(node_idx, edge_idx)
    a_row, einv_row = _tc_degrees(dvp, dep)
    acol = a_row.reshape(N, 1)
    einv_col = einv_row.reshape(M, 1)

    y1 = _tc_mm1(X, W1, b1, acol)
    ze = _tc_mid(_sc_spmm_call(y1, pairs_ne), einv_col)
    zv = _sc_spmm_call(ze, pairs_en)
    y2 = _tc_mm2(zv, W2, b2, acol)
    ze2 = _tc_mid(_sc_spmm_call(y2, pairs_ne), einv_col)
    zv2 = _sc_spmm_call(ze2, pairs_en)
    return _tc_final(zv2, acol)
